# Initial kernel scaffold; baseline (speedup 1.0000x reference)
import jax
import jax.numpy as jnp
from jax.experimental import pallas as pl


def kernel(X, batch_idx, label):
    raise NotImplementedError("write your pallas kernel here")


# speedup vs baseline: 212.2682x; 212.2682x over previous
import functools

import jax
import jax.numpy as jnp
import numpy as np
from jax import lax
from jax.experimental import pallas as pl
from jax.experimental.pallas import tpu as pltpu
from jax.experimental.pallas import tpu_sc as plsc

_N = 16384 * 200
_B = 16384

_info = plsc.get_sparse_core_info()
_NC = _info.num_cores
_NS = _info.num_subcores
_L = _info.num_lanes
_NW = _NC * _NS

_PER_W = _N // _NW
_CHUNK = 12800
_NCHUNK = _PER_W // _CHUNK
_UNROLL = 8
_VEC_STEPS = _CHUNK // (_L * _UNROLL)


def _sc_gather_dot(x, idx, label):
    mesh = plsc.VectorSubcoreMesh(core_axis_name="c", subcore_axis_name="s")

    @functools.partial(
        pl.kernel,
        mesh=mesh,
        out_type=jax.ShapeDtypeStruct((_NW, _L), jnp.float32),
        scratch_types=[
            pltpu.VMEM((_B,), jnp.float32),
            pltpu.VMEM((_CHUNK,), jnp.float32),
            pltpu.VMEM((_CHUNK,), jnp.int32),
            pltpu.VMEM((_L,), jnp.float32),
        ],
        compiler_params=pltpu.CompilerParams(needs_layout_passes=False),
    )
    def body(x_hbm, idx_hbm, label_hbm, out_hbm, label_v, x_v, idx_v, acc_v):
        wid = lax.axis_index("s") * _NC + lax.axis_index("c")
        base = wid * _PER_W
        pltpu.sync_copy(label_hbm, label_v)

        acc = jnp.zeros((_L,), jnp.float32)
        for c in range(_NCHUNK):
            off = base + c * _CHUNK
            pltpu.sync_copy(x_hbm.at[pl.ds(off, _CHUNK)], x_v)
            pltpu.sync_copy(idx_hbm.at[pl.ds(off, _CHUNK)], idx_v)

            def step(j, acc):
                for u in range(_UNROLL):
                    o = (j * _UNROLL + u) * _L
                    xv = x_v[pl.ds(o, _L)]
                    iv = idx_v[pl.ds(o, _L)]
                    g = plsc.load_gather(label_v, [iv])
                    acc = acc + xv * g
                return acc

            acc = lax.fori_loop(0, _VEC_STEPS, step, acc)

        acc_v[...] = acc
        pltpu.sync_copy(acc_v, out_hbm.at[wid])

    return body(x, idx, label)


_TC_ROWS = 3200
_TC_COLS = 1024
_TC_GRID = 8
_TC_BLK = _TC_ROWS // _TC_GRID


def _tc_dense_body(x_ref, o_ref):
    v = x_ref[...]
    val = jnp.maximum(v, 0.0) + jnp.log1p(jnp.exp(-jnp.abs(v)))
    s = jnp.sum(val, axis=0, keepdims=True)

    @pl.when(pl.program_id(0) == 0)
    def _init():
        o_ref[...] = s

    @pl.when(pl.program_id(0) != 0)
    def _acc():
        o_ref[...] += s


def _tc_dense_sum(x2):
    return pl.pallas_call(
        _tc_dense_body,
        grid=(_TC_GRID,),
        in_specs=[pl.BlockSpec((_TC_BLK, _TC_COLS), lambda i: (i, 0))],
        out_specs=pl.BlockSpec((1, _TC_COLS), lambda i: (0, 0)),
        out_shape=jax.ShapeDtypeStruct((1, _TC_COLS), jnp.float32),
    )(x2)


def kernel(X, batch_idx, label):
    x_flat = X.reshape(_N)
    sc_parts = _sc_gather_dot(x_flat, batch_idx.astype(jnp.int32), label)
    tc_parts = _tc_dense_sum(X.reshape(_TC_ROWS, _TC_COLS))
    total = jnp.sum(tc_parts) - jnp.sum(sc_parts)
    return total * np.float32(1.0 / _N)


# speedup vs baseline: 581.5682x; 2.7398x over previous
import functools

import jax
import jax.numpy as jnp
import numpy as np
from jax import lax
from jax.experimental import pallas as pl
from jax.experimental.pallas import tpu as pltpu
from jax.experimental.pallas import tpu_sc as plsc

_N = 16384 * 200
_B = 16384

_info = plsc.get_sparse_core_info()
_NC = _info.num_cores
_NS = _info.num_subcores
_L = _info.num_lanes
_NW = _NC * _NS

_EPW = _N // _NW
_CHUNKE = 20480
_NCHUNK = _EPW // _CHUNKE
_UNROLL = 8
_STEPS = _CHUNKE // (_L * _UNROLL)


def _sc_gather_dot(xf, idx, label):
    mesh = plsc.VectorSubcoreMesh(core_axis_name="c", subcore_axis_name="s")

    @functools.partial(
        pl.kernel,
        mesh=mesh,
        out_type=jax.ShapeDtypeStruct((_NW, _L), jnp.float32),
        scratch_types=[
            pltpu.VMEM((_B,), jnp.float32),
            pltpu.VMEM((_CHUNKE,), jnp.float32),
            pltpu.VMEM((_CHUNKE,), jnp.int32),
            pltpu.VMEM((_L,), jnp.float32),
        ],
        compiler_params=pltpu.CompilerParams(needs_layout_passes=False),
    )
    def body(x_hbm, idx_hbm, label_hbm, out_hbm, label_v, x_v, idx_v, acc_v):
        wid = lax.axis_index("s") * _NC + lax.axis_index("c")
        ebase = wid * _EPW
        pltpu.sync_copy(label_hbm, label_v)

        acc = jnp.zeros((_L,), jnp.float32)
        for c in range(_NCHUNK):
            pltpu.sync_copy(x_hbm.at[pl.ds(ebase + c * _CHUNKE, _CHUNKE)], x_v)
            pltpu.sync_copy(
                idx_hbm.at[pl.ds(ebase + c * _CHUNKE, _CHUNKE)], idx_v)

            def step(r, acc):
                for u in range(_UNROLL):
                    o = r * _L * _UNROLL + u * _L
                    xv = x_v[pl.ds(o, _L)]
                    iv = idx_v[pl.ds(o, _L)]
                    g = plsc.load_gather(label_v, [iv])
                    acc = acc + xv * g
                return acc

            acc = lax.fori_loop(0, _STEPS, step, acc)

        acc_v[...] = acc
        pltpu.sync_copy(acc_v, out_hbm.at[wid])

    return body(xf, idx, label)


_TC_GRID = 8
_TC_BLK = _N // _TC_GRID


def _tc_dense_body(x_ref, o_ref):
    v = x_ref[...].reshape(_TC_BLK // 128, 128)
    val = jnp.maximum(v, 0.0) + jnp.log1p(jnp.exp(-jnp.abs(v)))
    s = jnp.sum(val, axis=0, keepdims=True)

    @pl.when(pl.program_id(0) == 0)
    def _init():
        o_ref[...] = s

    @pl.when(pl.program_id(0) != 0)
    def _acc():
        o_ref[...] += s


def _tc_dense_sum(xf):
    return pl.pallas_call(
        _tc_dense_body,
        grid=(_TC_GRID,),
        in_specs=[pl.BlockSpec((_TC_BLK,), lambda i: (i,))],
        out_specs=pl.BlockSpec((1, 128), lambda i: (0, 0)),
        out_shape=jax.ShapeDtypeStruct((1, 128), jnp.float32),
    )(xf)


def kernel(X, batch_idx, label):
    xf = X.reshape(_N)
    sc_parts = _sc_gather_dot(xf, batch_idx.astype(jnp.int32), label)
    tc_parts = _tc_dense_sum(xf)
    total = jnp.sum(tc_parts) - jnp.sum(sc_parts)
    return total * np.float32(1.0 / _N)


# speedup vs baseline: 732.0112x; 1.2587x over previous
import functools

import jax
import jax.numpy as jnp
import numpy as np
from jax import lax
from jax.experimental import pallas as pl
from jax.experimental.pallas import tpu as pltpu
from jax.experimental.pallas import tpu_sc as plsc

_N = 16384 * 200
_B = 16384

_info = plsc.get_sparse_core_info()
_NC = _info.num_cores
_NS = _info.num_subcores
_L = _info.num_lanes
_NW = _NC * _NS

_EPW = _N // _NW
_CHUNKE = 20480
_NCHUNK = _EPW // _CHUNKE
_UNROLL = 16
_STEPS = _CHUNKE // (_L * _UNROLL)


def _sc_gather_dot(xf, idx, label):
    mesh = plsc.VectorSubcoreMesh(core_axis_name="c", subcore_axis_name="s")

    @functools.partial(
        pl.kernel,
        mesh=mesh,
        out_type=jax.ShapeDtypeStruct((_NW, _L), jnp.float32),
        scratch_types=[
            pltpu.VMEM((_B,), jnp.float32),
            pltpu.VMEM((_CHUNKE,), jnp.float32),
            pltpu.VMEM((_CHUNKE,), jnp.float32),
            pltpu.VMEM((_CHUNKE,), jnp.int32),
            pltpu.VMEM((_CHUNKE,), jnp.int32),
            pltpu.VMEM((_L,), jnp.float32),
            pltpu.SemaphoreType.DMA,
            pltpu.SemaphoreType.DMA,
        ],
        compiler_params=pltpu.CompilerParams(needs_layout_passes=False),
    )
    def body(x_hbm, idx_hbm, label_hbm, out_hbm,
             label_v, x0_v, x1_v, i0_v, i1_v, acc_v, sem0, sem1):
        wid = lax.axis_index("s") * _NC + lax.axis_index("c")
        ebase = wid * _EPW
        xbufs = (x0_v, x1_v)
        ibufs = (i0_v, i1_v)
        sems = (sem0, sem1)
        pltpu.sync_copy(label_hbm, label_v)

        def start(c):
            src = pl.ds(ebase + c * _CHUNKE, _CHUNKE)
            b = c % 2
            return (pltpu.async_copy(x_hbm.at[src], xbufs[b], sems[b]),
                    pltpu.async_copy(idx_hbm.at[src], ibufs[b], sems[b]))

        pending = start(0)
        acc = jnp.zeros((_L,), jnp.float32)
        for c in range(_NCHUNK):
            for h in pending:
                h.wait()
            if c + 1 < _NCHUNK:
                pending = start(c + 1)
            x_v = xbufs[c % 2]
            idx_v = ibufs[c % 2]

            def step(r, acc):
                for u in range(_UNROLL):
                    o = r * _L * _UNROLL + u * _L
                    xv = x_v[pl.ds(o, _L)]
                    iv = idx_v[pl.ds(o, _L)]
                    g = plsc.load_gather(label_v, [iv])
                    acc = acc + xv * g
                return acc

            acc = lax.fori_loop(0, _STEPS, step, acc)

        acc_v[...] = acc
        pltpu.sync_copy(acc_v, out_hbm.at[wid])

    return body(xf, idx, label)


_TC_GRID = 8
_TC_BLK = _N // _TC_GRID


def _tc_dense_body(x_ref, o_ref):
    v = x_ref[...].reshape(_TC_BLK // 128, 128)
    val = jnp.maximum(v, 0.0) + jnp.log1p(jnp.exp(-jnp.abs(v)))
    s = jnp.sum(val, axis=0, keepdims=True)

    @pl.when(pl.program_id(0) == 0)
    def _init():
        o_ref[...] = s

    @pl.when(pl.program_id(0) != 0)
    def _acc():
        o_ref[...] += s


def _tc_dense_sum(xf):
    return pl.pallas_call(
        _tc_dense_body,
        grid=(_TC_GRID,),
        in_specs=[pl.BlockSpec((_TC_BLK,), lambda i: (i,))],
        out_specs=pl.BlockSpec((1, 128), lambda i: (0, 0)),
        out_shape=jax.ShapeDtypeStruct((1, 128), jnp.float32),
    )(xf)


def kernel(X, batch_idx, label):
    xf = X.reshape(_N)
    sc_parts = _sc_gather_dot(xf, batch_idx.astype(jnp.int32), label)
    tc_parts = _tc_dense_sum(xf)
    total = jnp.sum(tc_parts) - jnp.sum(sc_parts)
    return total * np.float32(1.0 / _N)
